# Initial kernel scaffold; baseline (speedup 1.0000x reference)
import jax
import jax.numpy as jnp
from jax.experimental import pallas as pl


def kernel(x, edge_index, W0, W1, W2, b2, g0, be0, g1, be1):
    raise NotImplementedError("write your pallas kernel here")


# speedup vs baseline: 5.3748x; 5.3748x over previous
import functools

import jax
import jax.numpy as jnp
from jax import lax
from jax.experimental import pallas as pl
from jax.experimental.pallas import tpu as pltpu
from jax.experimental.pallas import tpu_sc as plsc

NN = 10000
EE = 320000
D_IN = 128
HID = 256
NCLS = 40
EPS = 1e-5

NC, NS = 2, 16
NW = NC * NS
N_PAD = 10240
ROWS_PT = N_PAD // NS
E_PAD = 327680
BLK = 128
CH = 128
JUNK = N_PAD - NN
F3 = 48

_mesh = plsc.VectorSubcoreMesh(core_axis_name="c", subcore_axis_name="s")


def _fill(ref, rows, cols, value):
    npc = cols // 16

    def body(i, _):
        r = i // npc
        col = (i % npc) * 16
        ref[r, pl.ds(col, 16)] = jnp.full((16,), value, jnp.float32)
        return 0

    lax.fori_loop(0, rows * npc, body, 0)


def _fill1d(ref, n, value):
    def body(i, _):
        ref[pl.ds(i * 16, 16)] = jnp.full((16,), value, jnp.float32)
        return 0

    lax.fori_loop(0, n // 16, body, 0)


@functools.partial(
    pl.kernel,
    out_type=jax.ShapeDtypeStruct((NC, N_PAD), jnp.float32),
    mesh=_mesh,
    scratch_types=[
        pltpu.VMEM((BLK,), jnp.int32),
        pltpu.VMEM((BLK,), jnp.float32),
        pltpu.VMEM((ROWS_PT,), jnp.float32),
        pltpu.VMEM_SHARED((N_PAD,), jnp.float32),
    ],
)
def _hist(src, dst, out, idx_v, ones_v, buf_v, acc):
    c = lax.axis_index("c")
    s = lax.axis_index("s")
    _fill1d(ones_v, BLK, 1.0)
    _fill1d(buf_v, ROWS_PT, 0.0)
    pltpu.sync_copy(buf_v, acc.at[pl.ds(s * ROWS_PT, ROWS_PT)])
    plsc.subcore_barrier()

    nblk = (E_PAD // NS) // BLK

    def step(i, _):
        base = s * (E_PAD // NS) + i * BLK

        @pl.when(c == 0)
        def _():
            pltpu.sync_copy(src.at[pl.ds(base, BLK)], idx_v)

        @pl.when(c == 1)
        def _():
            pltpu.sync_copy(dst.at[pl.ds(base, BLK)], idx_v)

        pltpu.sync_copy(ones_v, acc.at[idx_v], add=True)
        return 0

    lax.fori_loop(0, nblk, step, 0)
    plsc.subcore_barrier()
    pltpu.sync_copy(acc.at[pl.ds(s * ROWS_PT, ROWS_PT)], buf_v)
    pltpu.sync_copy(buf_v, out.at[c, pl.ds(s * ROWS_PT, ROWS_PT)])


def _make_agg_edge(F):
    @functools.partial(
        pl.kernel,
        out_type=jax.ShapeDtypeStruct((NC, N_PAD, F), jnp.float32),
        mesh=_mesh,
        scratch_types=[
            pltpu.VMEM((BLK,), jnp.int32),
            pltpu.VMEM((BLK,), jnp.int32),
            pltpu.VMEM((BLK, F), jnp.float32),
            pltpu.VMEM((CH, F), jnp.float32),
            pltpu.VMEM_SHARED((N_PAD, F), jnp.float32),
            pltpu.SemaphoreType.DMA,
        ],
        compiler_params=pltpu.CompilerParams(use_tc_tiling_on_sc=(F % 128 == 0)),
    )
    def agg(feats, src, dst, out, sidx_v, didx_v, rows_v, buf_v, acc, sem):
        c = lax.axis_index("c")
        s = lax.axis_index("s")
        wid = s * NC + c
        _fill(buf_v, CH, F, 0.0)

        def zacc(k, _):
            pltpu.sync_copy(buf_v, acc.at[pl.ds(s * ROWS_PT + k * CH, CH), :])
            return 0

        lax.fori_loop(0, ROWS_PT // CH, zacc, 0)
        plsc.subcore_barrier()

        epw = E_PAD // NW
        nblk = epw // BLK

        def step(i, _):
            base = wid * epw + i * BLK
            pltpu.sync_copy(src.at[pl.ds(base, BLK)], sidx_v)
            cp = pltpu.async_copy(feats.at[sidx_v], rows_v, sem)
            pltpu.sync_copy(dst.at[pl.ds(base, BLK)], didx_v)
            cp.wait()
            pltpu.sync_copy(rows_v, acc.at[didx_v], add=True)
            return 0

        lax.fori_loop(0, nblk, step, 0)
        plsc.subcore_barrier()

        def co(k, _):
            r0 = s * ROWS_PT + k * CH
            pltpu.sync_copy(acc.at[pl.ds(r0, CH), :], buf_v)
            pltpu.sync_copy(buf_v, out.at[c, pl.ds(r0, CH), :])
            return 0

        lax.fori_loop(0, ROWS_PT // CH, co, 0)

    return agg


_agg_e128 = _make_agg_edge(D_IN)
_agg_e48 = _make_agg_edge(F3)


@functools.partial(
    pl.kernel,
    out_type=jax.ShapeDtypeStruct((NC, N_PAD, 128), jnp.float32),
    mesh=_mesh,
    scratch_types=[
        pltpu.VMEM((BLK,), jnp.int32),
        pltpu.VMEM((BLK,), jnp.int32),
        pltpu.VMEM((BLK, 128), jnp.float32),
        pltpu.VMEM((CH, 128), jnp.float32),
        pltpu.VMEM_SHARED((N_PAD, 128), jnp.float32),
        pltpu.SemaphoreType.DMA,
    ],
)
def _agg_feat(ha, hb, src, dst, out, sidx_v, didx_v, rows_v, buf_v, acc, sem):
    c = lax.axis_index("c")
    s = lax.axis_index("s")
    _fill(buf_v, CH, 128, 0.0)

    def zacc(k, _):
        pltpu.sync_copy(buf_v, acc.at[pl.ds(s * ROWS_PT + k * CH, CH), :])
        return 0

    lax.fori_loop(0, ROWS_PT // CH, zacc, 0)
    plsc.subcore_barrier()

    ept = E_PAD // NS
    nblk = ept // BLK

    def step(i, _):
        base = s * ept + i * BLK
        pltpu.sync_copy(src.at[pl.ds(base, BLK)], sidx_v)

        @pl.when(c == 0)
        def _():
            pltpu.async_copy(ha.at[sidx_v], rows_v, sem).wait()

        @pl.when(c == 1)
        def _():
            pltpu.async_copy(hb.at[sidx_v], rows_v, sem).wait()

        pltpu.sync_copy(dst.at[pl.ds(base, BLK)], didx_v)
        pltpu.sync_copy(rows_v, acc.at[didx_v], add=True)
        return 0

    lax.fori_loop(0, nblk, step, 0)
    plsc.subcore_barrier()

    def co(k, _):
        r0 = s * ROWS_PT + k * CH
        pltpu.sync_copy(acc.at[pl.ds(r0, CH), :], buf_v)
        pltpu.sync_copy(buf_v, out.at[c, pl.ds(r0, CH), :])
        return 0

    lax.fori_loop(0, ROWS_PT // CH, co, 0)


def _t0_body(x_ref, cnt_ref, scales_ref, xs_ref):
    sc = lax.rsqrt(jnp.maximum(cnt_ref[...], 1.0))
    scales_ref[...] = sc
    xs_ref[:NN, :] = x_ref[...] * sc[:, 0:1]
    xs_ref[NN:, :] = jnp.zeros((JUNK, D_IN), jnp.float32)


_t0 = pl.pallas_call(
    _t0_body,
    out_shape=(
        jax.ShapeDtypeStruct((NN, 2), jnp.float32),
        jax.ShapeDtypeStruct((N_PAD, D_IN), jnp.float32),
    ),
)


def _t1_body(p_ref, sc_ref, w0_ref, w1_ref, g0_ref, be0_ref, ha_ref, hb_ref):
    agg = p_ref[0, :NN, :] + p_ref[1, :NN, :]
    dis = sc_ref[...]
    h = jnp.dot(agg, w0_ref[...], preferred_element_type=jnp.float32)
    h = h * dis[:, 1:2]
    mu = jnp.mean(h, axis=0, keepdims=True)
    var = jnp.mean((h - mu) ** 2, axis=0, keepdims=True)
    hn = (h - mu) * lax.rsqrt(var + EPS) * g0_ref[...][None, :] + be0_ref[...][None, :]
    hn = jnp.maximum(hn, 0.0) * dis[:, 0:1]
    h2 = jnp.dot(hn, w1_ref[...], preferred_element_type=jnp.float32)
    ha_ref[:NN, :] = h2[:, :128]
    ha_ref[NN:, :] = jnp.zeros((JUNK, 128), jnp.float32)
    hb_ref[:NN, :] = h2[:, 128:]
    hb_ref[NN:, :] = jnp.zeros((JUNK, 128), jnp.float32)


_t1 = pl.pallas_call(
    _t1_body,
    out_shape=(
        jax.ShapeDtypeStruct((N_PAD, 128), jnp.float32),
        jax.ShapeDtypeStruct((N_PAD, 128), jnp.float32),
    ),
)


def _t2_body(q_ref, sc_ref, w2_ref, g1_ref, be1_ref, f3_ref):
    agg = jnp.concatenate([q_ref[0, :NN, :], q_ref[1, :NN, :]], axis=1)
    dis = sc_ref[...]
    h = agg * dis[:, 1:2]
    mu = jnp.mean(h, axis=0, keepdims=True)
    var = jnp.mean((h - mu) ** 2, axis=0, keepdims=True)
    hn = (h - mu) * lax.rsqrt(var + EPS) * g1_ref[...][None, :] + be1_ref[...][None, :]
    hn = jnp.maximum(hn, 0.0) * dis[:, 0:1]
    h3 = jnp.dot(hn, w2_ref[...], preferred_element_type=jnp.float32)
    f3_ref[:NN, :NCLS] = h3
    f3_ref[:NN, NCLS:] = jnp.zeros((NN, F3 - NCLS), jnp.float32)
    f3_ref[NN:, :] = jnp.zeros((JUNK, F3), jnp.float32)


_t2 = pl.pallas_call(
    _t2_body,
    out_shape=jax.ShapeDtypeStruct((N_PAD, F3), jnp.float32),
)


def _t3_body(r_ref, sc_ref, b2_ref, out_ref):
    agg = r_ref[0, :NN, :NCLS] + r_ref[1, :NN, :NCLS]
    dis = sc_ref[...]
    out_ref[...] = agg * dis[:, 1:2] + b2_ref[...][None, :]


_t3 = pl.pallas_call(
    _t3_body,
    out_shape=jax.ShapeDtypeStruct((NN, NCLS), jnp.float32),
)


def kernel(x, edge_index, W0, W1, W2, b2, g0, be0, g1, be1):
    src = edge_index[0]
    dst = edge_index[1]
    pad = (jnp.arange(E_PAD - EE, dtype=jnp.int32) % JUNK) + NN
    srcp = jnp.concatenate([src, pad])
    dstp = jnp.concatenate([dst, pad])

    cnts = _hist(srcp, dstp)
    cnt_t = cnts[:, :NN].T
    scales, xs = _t0(x, cnt_t)
    p1 = _agg_e128(xs, srcp, dstp)
    ha, hb = _t1(p1, scales, W0, W1, g0, be0)
    q2 = _agg_feat(ha, hb, srcp, dstp)
    f3 = _t2(q2, scales, W2, g1, be1)
    r3 = _agg_e48(f3, srcp, dstp)
    return _t3(r3, scales, b2)
